# baseline (speedup 1.0000x reference)
import functools

import jax
import jax.numpy as jnp
from jax import lax
from jax.experimental import pallas as pl
from jax.experimental.pallas import tpu as pltpu
from jax.experimental.pallas import tpu_sc as plsc

_B, _NPER, _DIN, _DH, _DM, _PSTEP = 100, 100, 128, 64, 32, 3
_N = _B * _NPER
_NPAD = 10112
_E = 320000
_S = 64
_FEAT = 1 + _S + _NPER * _DH
_CHUNK = 128
_CHUNKS = 80
_UNROLL = 4
_NTILES = 32
_EPAD = _NTILES * _CHUNKS * _CHUNK
_RPT = _NPAD // 16
_NBLK = 4
_RBLK = _NPAD // _NBLK


def _embed_body(x_ref, we_ref, be_ref, ws_ref, wd_ref, bm_ref,
                h_ref, ms_ref, md_ref):
    h = jnp.tanh(
        jnp.dot(x_ref[...], we_ref[...], preferred_element_type=jnp.float32)
        + be_ref[...])
    h_ref[...] = h
    ms_ref[...] = jnp.dot(h, ws_ref[...], preferred_element_type=jnp.float32)
    md_ref[...] = (
        jnp.dot(h, wd_ref[...], preferred_element_type=jnp.float32)
        + bm_ref[...])


def _tc_embed(x_pad, we, be2, ws, wd, bm2):
    return pl.pallas_call(
        _embed_body,
        grid=(_NBLK,),
        in_specs=[
            pl.BlockSpec((_RBLK, _DIN), lambda i: (i, 0)),
            pl.BlockSpec((_DIN, _DH), lambda i: (0, 0)),
            pl.BlockSpec((1, _DH), lambda i: (0, 0)),
            pl.BlockSpec((_DH, _DM), lambda i: (0, 0)),
            pl.BlockSpec((_DH, _DM), lambda i: (0, 0)),
            pl.BlockSpec((1, _DM), lambda i: (0, 0)),
        ],
        out_specs=[
            pl.BlockSpec((_RBLK, _DH), lambda i: (i, 0)),
            pl.BlockSpec((_RBLK, _DM), lambda i: (i, 0)),
            pl.BlockSpec((_RBLK, _DM), lambda i: (i, 0)),
        ],
        out_shape=[
            jax.ShapeDtypeStruct((_NPAD, _DH), jnp.float32),
            jax.ShapeDtypeStruct((_NPAD, _DM), jnp.float32),
            jax.ShapeDtypeStruct((_NPAD, _DM), jnp.float32),
        ],
    )(x_pad, we, be2, ws, wd, bm2)


def _update_body(h_ref, agg_ref, wu1_ref, wu2_ref, bu_ref, ws_ref, wd_ref,
                 bm_ref, hn_ref, ms_ref, md_ref):
    agg = agg_ref[0] + agg_ref[1]
    hn = jnp.tanh(
        jnp.dot(h_ref[...], wu1_ref[...], preferred_element_type=jnp.float32)
        + jnp.dot(agg, wu2_ref[...], preferred_element_type=jnp.float32)
        + bu_ref[...])
    hn_ref[...] = hn
    ms_ref[...] = jnp.dot(hn, ws_ref[...], preferred_element_type=jnp.float32)
    md_ref[...] = (
        jnp.dot(hn, wd_ref[...], preferred_element_type=jnp.float32)
        + bm_ref[...])


def _tc_update(h, agg2, wu1, wu2, bu2, ws, wd, bm2):
    return pl.pallas_call(
        _update_body,
        grid=(_NBLK,),
        in_specs=[
            pl.BlockSpec((_RBLK, _DH), lambda i: (i, 0)),
            pl.BlockSpec((2, _RBLK, _DM), lambda i: (0, i, 0)),
            pl.BlockSpec((_DH, _DH), lambda i: (0, 0)),
            pl.BlockSpec((_DM, _DH), lambda i: (0, 0)),
            pl.BlockSpec((1, _DH), lambda i: (0, 0)),
            pl.BlockSpec((_DH, _DM), lambda i: (0, 0)),
            pl.BlockSpec((_DH, _DM), lambda i: (0, 0)),
            pl.BlockSpec((1, _DM), lambda i: (0, 0)),
        ],
        out_specs=[
            pl.BlockSpec((_RBLK, _DH), lambda i: (i, 0)),
            pl.BlockSpec((_RBLK, _DM), lambda i: (i, 0)),
            pl.BlockSpec((_RBLK, _DM), lambda i: (i, 0)),
        ],
        out_shape=[
            jax.ShapeDtypeStruct((_NPAD, _DH), jnp.float32),
            jax.ShapeDtypeStruct((_NPAD, _DM), jnp.float32),
            jax.ShapeDtypeStruct((_NPAD, _DM), jnp.float32),
        ],
    )(h, agg2, wu1, wu2, bu2, ws, wd, bm2)


def _edge_body(ms_hbm, md_hbm, src_hbm, dst_hbm, out_hbm,
               src_v, dst_v, ga0, gb0, ga1, gb1, mb0, mb1, stage, agg_s,
               sg0, sg1, ss0, ss1):
    cid = lax.axis_index("c")
    sid = lax.axis_index("s")
    wid = cid * 16 + sid
    pltpu.sync_copy(src_hbm.at[wid], src_v)
    pltpu.sync_copy(dst_hbm.at[wid], dst_v)
    zero = jnp.zeros((16,), jnp.float32)

    def _zrow(r, c):
        stage[r, 0:16] = zero
        stage[r, 16:32] = zero
        return c

    lax.fori_loop(0, _RPT, _zrow, 0)
    pltpu.sync_copy(stage, agg_s.at[pl.ds(sid * _RPT, _RPT)])
    plsc.subcore_barrier()

    def _fire(j, ga, gb, sem):
        pltpu.async_copy(ms_hbm.at[src_v.at[j]], ga, sem)
        pltpu.async_copy(md_hbm.at[dst_v.at[j]], gb, sem)

    def _drain(ga, gb, sem):
        pltpu.make_async_copy(ms_hbm.at[src_v.at[0]], ga, sem).wait()
        pltpu.make_async_copy(md_hbm.at[dst_v.at[0]], gb, sem).wait()

    def _compute(ga, gb, mb):
        def _rows(r4, c2):
            for k in range(_UNROLL):
                r = r4 * _UNROLL + k
                mb[r, 0:16] = jnp.maximum(ga[r, 0:16] + gb[r, 0:16], 0.0)
                mb[r, 16:32] = jnp.maximum(ga[r, 16:32] + gb[r, 16:32], 0.0)
            return c2

        lax.fori_loop(0, _CHUNK // _UNROLL, _rows, 0)

    def _sfire(j, mb, sem):
        pltpu.async_copy(mb, agg_s.at[dst_v.at[j]], sem, add=True)

    def _sdrain(mb, sem):
        pltpu.make_async_copy(mb, agg_s.at[dst_v.at[0]], sem).wait()

    _fire(0, ga0, gb0, sg0)
    _fire(1, ga1, gb1, sg1)
    _drain(ga0, gb0, sg0)
    _compute(ga0, gb0, mb0)
    _fire(2, ga0, gb0, sg0)
    _sfire(0, mb0, ss0)
    _drain(ga1, gb1, sg1)
    _compute(ga1, gb1, mb1)
    _fire(3, ga1, gb1, sg1)
    _sfire(1, mb1, ss1)

    def _body(i, c):
        j0 = i * 2
        _drain(ga0, gb0, sg0)
        _sdrain(mb0, ss0)
        _compute(ga0, gb0, mb0)
        _fire(j0 + 2, ga0, gb0, sg0)
        _sfire(j0, mb0, ss0)
        _drain(ga1, gb1, sg1)
        _sdrain(mb1, ss1)
        _compute(ga1, gb1, mb1)
        _fire(j0 + 3, ga1, gb1, sg1)
        _sfire(j0 + 1, mb1, ss1)
        return c

    lax.fori_loop(1, _CHUNKS // 2 - 1, _body, 0)
    _drain(ga0, gb0, sg0)
    _sdrain(mb0, ss0)
    _compute(ga0, gb0, mb0)
    _sfire(_CHUNKS - 2, mb0, ss0)
    _drain(ga1, gb1, sg1)
    _sdrain(mb1, ss1)
    _compute(ga1, gb1, mb1)
    _sfire(_CHUNKS - 1, mb1, ss1)
    _sdrain(mb0, ss0)
    _sdrain(mb1, ss1)
    plsc.subcore_barrier()
    pltpu.sync_copy(agg_s.at[pl.ds(sid * _RPT, _RPT)], stage)
    pltpu.sync_copy(stage, out_hbm.at[cid, pl.ds(sid * _RPT, _RPT), :])


_edge_kernel = functools.partial(
    pl.kernel,
    out_type=jax.ShapeDtypeStruct((2, _NPAD, _DM), jnp.float32),
    mesh=plsc.VectorSubcoreMesh(core_axis_name="c", subcore_axis_name="s"),
    compiler_params=pltpu.CompilerParams(use_tc_tiling_on_sc=False),
    scratch_types=[
        pltpu.VMEM((_CHUNKS, _CHUNK), jnp.int32),
        pltpu.VMEM((_CHUNKS, _CHUNK), jnp.int32),
        pltpu.VMEM((_CHUNK, _DM), jnp.float32),
        pltpu.VMEM((_CHUNK, _DM), jnp.float32),
        pltpu.VMEM((_CHUNK, _DM), jnp.float32),
        pltpu.VMEM((_CHUNK, _DM), jnp.float32),
        pltpu.VMEM((_CHUNK, _DM), jnp.float32),
        pltpu.VMEM((_CHUNK, _DM), jnp.float32),
        pltpu.VMEM((_RPT, _DM), jnp.float32),
        pltpu.VMEM_SHARED((_NPAD, _DM), jnp.float32),
        pltpu.SemaphoreType.DMA,
        pltpu.SemaphoreType.DMA,
        pltpu.SemaphoreType.DMA,
        pltpu.SemaphoreType.DMA,
    ],
)(_edge_body)


def _head1_body(x_ref, mean_ref, var_ref, w1_ref, b1_ref, a1_ref):
    xn = (x_ref[...] - mean_ref[...]) * jax.lax.rsqrt(var_ref[...] + 1e-8)
    xn = jnp.clip(xn, -5.0, 5.0)
    a1_ref[...] = jnp.tanh(
        jnp.dot(xn, w1_ref[...], preferred_element_type=jnp.float32)
        + b1_ref[...])


def _tc_head1(x, mean2, var2, w1, b12):
    return pl.pallas_call(
        _head1_body,
        grid=(4,),
        in_specs=[
            pl.BlockSpec((_B, _FEAT), lambda j: (0, 0)),
            pl.BlockSpec((1, _FEAT), lambda j: (0, 0)),
            pl.BlockSpec((1, _FEAT), lambda j: (0, 0)),
            pl.BlockSpec((_FEAT, 128), lambda j: (0, j)),
            pl.BlockSpec((1, 128), lambda j: (0, j)),
        ],
        out_specs=pl.BlockSpec((_B, 128), lambda j: (0, j)),
        out_shape=jax.ShapeDtypeStruct((_B, 512), jnp.float32),
    )(x, mean2, var2, w1, b12)


def _head2_body(a1_ref, w2_ref, b2_ref, wv_ref, bv_ref, v_ref):
    a2 = jnp.tanh(
        jnp.dot(a1_ref[...], w2_ref[...], preferred_element_type=jnp.float32)
        + b2_ref[...])
    v_ref[...] = (
        jnp.dot(a2, wv_ref[...], preferred_element_type=jnp.float32)
        + bv_ref[...])


def _tc_head2(a1, w2, b22, wv, bv2):
    return pl.pallas_call(
        _head2_body,
        out_shape=jax.ShapeDtypeStruct((_B, 1), jnp.float32),
    )(a1, w2, b22, wv, bv2)


def kernel(stage_ind, scale_state, sim_obs, edges, W_embed, b_embed, W_msg,
           b_msg, W_upd, b_upd, norm_mean, norm_var, W1, b1, W2, b2, Wv, bv):
    f32 = jnp.float32
    ws = W_msg[:_DH]
    wd = W_msg[_DH:]
    wu1 = W_upd[:_DH]
    wu2 = W_upd[_DH:]
    be2 = b_embed.reshape(1, _DH)
    bm2 = b_msg.reshape(1, _DM)
    bu2 = b_upd.reshape(1, _DH)

    x_pad = jnp.pad(sim_obs, ((0, _NPAD - _N), (0, 0)))
    src = jnp.pad(edges[0], (0, _EPAD - _E), constant_values=_N)
    dst = jnp.pad(edges[1], (0, _EPAD - _E), constant_values=_N)
    src = src.reshape(_NTILES, _CHUNKS, _CHUNK)
    dst = dst.reshape(_NTILES, _CHUNKS, _CHUNK)

    h, ms, md = _tc_embed(x_pad, W_embed, be2, ws, wd, bm2)
    for _ in range(_PSTEP):
        agg2 = _edge_kernel(ms, md, src, dst)
        h, ms, md = _tc_update(h, agg2, wu1, wu2, bu2, ws, wd, bm2)

    gnn = h[:_N].reshape(_B, _NPER * _DH)
    x = jnp.concatenate([stage_ind, scale_state, gnn], axis=1)
    a1 = _tc_head1(x, norm_mean.reshape(1, _FEAT), norm_var.reshape(1, _FEAT),
                   W1, b1.reshape(1, 512))
    value = _tc_head2(a1, W2, b2.reshape(1, 256), Wv,
                      bv.reshape(1, 1).astype(f32))
    return value


# speedup vs baseline: 1.2613x; 1.2613x over previous
import functools

import jax
import jax.numpy as jnp
from jax import lax
from jax.experimental import pallas as pl
from jax.experimental.pallas import tpu as pltpu
from jax.experimental.pallas import tpu_sc as plsc

_B, _NPER, _DIN, _DH, _DM, _PSTEP = 100, 100, 128, 64, 32, 3
_N = _B * _NPER
_NPAD = 10112
_E = 320000
_S = 64
_FEAT = 1 + _S + _NPER * _DH
_CHUNK = 128
_CHUNKS = 80
_UNROLL = 4
_NTILES = 32
_EPAD = _NTILES * _CHUNKS * _CHUNK
_RPT = _NPAD // 16
_NBLK = 4
_RBLK = _NPAD // _NBLK
_AGG_PERM = tuple(range(0, _DM, 2)) + tuple(range(1, _DM, 2))


def _embed_body(x_ref, we_ref, be_ref, ws_ref, wd_ref, bm_ref,
                h_ref, ms_ref, md_ref):
    h = jnp.tanh(
        jnp.dot(x_ref[...], we_ref[...], preferred_element_type=jnp.float32)
        + be_ref[...])
    h_ref[...] = h
    ms_ref[...] = jnp.dot(
        h, ws_ref[...], preferred_element_type=jnp.float32
    ).astype(jnp.bfloat16)
    md_ref[...] = (
        jnp.dot(h, wd_ref[...], preferred_element_type=jnp.float32)
        + bm_ref[...]).astype(jnp.bfloat16)


def _tc_embed(x_pad, we, be2, ws, wd, bm2):
    return pl.pallas_call(
        _embed_body,
        grid=(_NBLK,),
        in_specs=[
            pl.BlockSpec((_RBLK, _DIN), lambda i: (i, 0)),
            pl.BlockSpec((_DIN, _DH), lambda i: (0, 0)),
            pl.BlockSpec((1, _DH), lambda i: (0, 0)),
            pl.BlockSpec((_DH, _DM), lambda i: (0, 0)),
            pl.BlockSpec((_DH, _DM), lambda i: (0, 0)),
            pl.BlockSpec((1, _DM), lambda i: (0, 0)),
        ],
        out_specs=[
            pl.BlockSpec((_RBLK, _DH), lambda i: (i, 0)),
            pl.BlockSpec((_RBLK, _DM), lambda i: (i, 0)),
            pl.BlockSpec((_RBLK, _DM), lambda i: (i, 0)),
        ],
        out_shape=[
            jax.ShapeDtypeStruct((_NPAD, _DH), jnp.float32),
            jax.ShapeDtypeStruct((_NPAD, _DM), jnp.bfloat16),
            jax.ShapeDtypeStruct((_NPAD, _DM), jnp.bfloat16),
        ],
    )(x_pad, we, be2, ws, wd, bm2)


def _update_body(h_ref, agg_ref, wu1_ref, wu2_ref, bu_ref, ws_ref, wd_ref,
                 bm_ref, hn_ref, ms_ref, md_ref):
    agg = agg_ref[0] + agg_ref[1]
    hn = jnp.tanh(
        jnp.dot(h_ref[...], wu1_ref[...], preferred_element_type=jnp.float32)
        + jnp.dot(agg, wu2_ref[...], preferred_element_type=jnp.float32)
        + bu_ref[...])
    hn_ref[...] = hn
    ms_ref[...] = jnp.dot(
        hn, ws_ref[...], preferred_element_type=jnp.float32
    ).astype(jnp.bfloat16)
    md_ref[...] = (
        jnp.dot(hn, wd_ref[...], preferred_element_type=jnp.float32)
        + bm_ref[...]).astype(jnp.bfloat16)


def _tc_update(h, agg2, wu1, wu2, bu2, ws, wd, bm2):
    return pl.pallas_call(
        _update_body,
        grid=(_NBLK,),
        in_specs=[
            pl.BlockSpec((_RBLK, _DH), lambda i: (i, 0)),
            pl.BlockSpec((2, _RBLK, _DM), lambda i: (0, i, 0)),
            pl.BlockSpec((_DH, _DH), lambda i: (0, 0)),
            pl.BlockSpec((_DM, _DH), lambda i: (0, 0)),
            pl.BlockSpec((1, _DH), lambda i: (0, 0)),
            pl.BlockSpec((_DH, _DM), lambda i: (0, 0)),
            pl.BlockSpec((_DH, _DM), lambda i: (0, 0)),
            pl.BlockSpec((1, _DM), lambda i: (0, 0)),
        ],
        out_specs=[
            pl.BlockSpec((_RBLK, _DH), lambda i: (i, 0)),
            pl.BlockSpec((_RBLK, _DM), lambda i: (i, 0)),
            pl.BlockSpec((_RBLK, _DM), lambda i: (i, 0)),
        ],
        out_shape=[
            jax.ShapeDtypeStruct((_NPAD, _DH), jnp.float32),
            jax.ShapeDtypeStruct((_NPAD, _DM), jnp.bfloat16),
            jax.ShapeDtypeStruct((_NPAD, _DM), jnp.bfloat16),
        ],
    )(h, agg2, wu1, wu2, bu2, ws, wd, bm2)


def _edge_body(ms_hbm, md_hbm, src_hbm, dst_hbm, out_hbm,
               src_v, dst_v, ga0, gb0, ga1, gb1, mb0, mb1, stage, agg_s,
               sg0, sg1, ss0, ss1):
    cid = lax.axis_index("c")
    sid = lax.axis_index("s")
    wid = cid * 16 + sid
    pltpu.sync_copy(src_hbm.at[wid], src_v)
    pltpu.sync_copy(dst_hbm.at[wid], dst_v)
    zero = jnp.zeros((16,), jnp.float32)

    def _zrow(r, c):
        stage[r, 0:16] = zero
        stage[r, 16:32] = zero
        return c

    lax.fori_loop(0, _RPT, _zrow, 0)
    pltpu.sync_copy(stage, agg_s.at[pl.ds(sid * _RPT, _RPT)])
    plsc.subcore_barrier()

    def _fire(j, ga, gb, sem):
        pltpu.async_copy(ms_hbm.at[src_v.at[j]], ga, sem)
        pltpu.async_copy(md_hbm.at[dst_v.at[j]], gb, sem)

    def _drain(ga, gb, sem):
        pltpu.make_async_copy(ms_hbm.at[src_v.at[0]], ga, sem).wait()
        pltpu.make_async_copy(md_hbm.at[dst_v.at[0]], gb, sem).wait()

    def _compute(ga, gb, mb):
        zero_bf = jnp.zeros((32,), jnp.bfloat16)
        mask_hi = jnp.full((16,), -65536, jnp.int32)

        def _rows(r4, c2):
            for k in range(_UNROLL):
                r = r4 * _UNROLL + k
                m = jnp.maximum(ga[r, 0:32] + gb[r, 0:32], zero_bf)
                w = plsc.bitcast(m, jnp.int32)
                mb[r, 0:16] = plsc.bitcast(w << 16, jnp.float32)
                mb[r, 16:32] = plsc.bitcast(w & mask_hi, jnp.float32)
            return c2

        lax.fori_loop(0, _CHUNK // _UNROLL, _rows, 0)

    def _sfire(j, mb, sem):
        pltpu.async_copy(mb, agg_s.at[dst_v.at[j]], sem, add=True)

    def _sdrain(mb, sem):
        pltpu.make_async_copy(mb, agg_s.at[dst_v.at[0]], sem).wait()

    _fire(0, ga0, gb0, sg0)
    _fire(1, ga1, gb1, sg1)
    _drain(ga0, gb0, sg0)
    _compute(ga0, gb0, mb0)
    _fire(2, ga0, gb0, sg0)
    _sfire(0, mb0, ss0)
    _drain(ga1, gb1, sg1)
    _compute(ga1, gb1, mb1)
    _fire(3, ga1, gb1, sg1)
    _sfire(1, mb1, ss1)

    def _body(i, c):
        j0 = i * 2
        _drain(ga0, gb0, sg0)
        _sdrain(mb0, ss0)
        _compute(ga0, gb0, mb0)
        _fire(j0 + 2, ga0, gb0, sg0)
        _sfire(j0, mb0, ss0)
        _drain(ga1, gb1, sg1)
        _sdrain(mb1, ss1)
        _compute(ga1, gb1, mb1)
        _fire(j0 + 3, ga1, gb1, sg1)
        _sfire(j0 + 1, mb1, ss1)
        return c

    lax.fori_loop(1, _CHUNKS // 2 - 1, _body, 0)
    _drain(ga0, gb0, sg0)
    _sdrain(mb0, ss0)
    _compute(ga0, gb0, mb0)
    _sfire(_CHUNKS - 2, mb0, ss0)
    _drain(ga1, gb1, sg1)
    _sdrain(mb1, ss1)
    _compute(ga1, gb1, mb1)
    _sfire(_CHUNKS - 1, mb1, ss1)
    _sdrain(mb0, ss0)
    _sdrain(mb1, ss1)
    plsc.subcore_barrier()
    pltpu.sync_copy(agg_s.at[pl.ds(sid * _RPT, _RPT)], stage)
    pltpu.sync_copy(stage, out_hbm.at[cid, pl.ds(sid * _RPT, _RPT), :])


_edge_kernel = functools.partial(
    pl.kernel,
    out_type=jax.ShapeDtypeStruct((2, _NPAD, _DM), jnp.float32),
    mesh=plsc.VectorSubcoreMesh(core_axis_name="c", subcore_axis_name="s"),
    compiler_params=pltpu.CompilerParams(
        use_tc_tiling_on_sc=False, needs_layout_passes=False),
    scratch_types=[
        pltpu.VMEM((_CHUNKS, _CHUNK), jnp.int32),
        pltpu.VMEM((_CHUNKS, _CHUNK), jnp.int32),
        pltpu.VMEM((_CHUNK, _DM), jnp.bfloat16),
        pltpu.VMEM((_CHUNK, _DM), jnp.bfloat16),
        pltpu.VMEM((_CHUNK, _DM), jnp.bfloat16),
        pltpu.VMEM((_CHUNK, _DM), jnp.bfloat16),
        pltpu.VMEM((_CHUNK, _DM), jnp.float32),
        pltpu.VMEM((_CHUNK, _DM), jnp.float32),
        pltpu.VMEM((_RPT, _DM), jnp.float32),
        pltpu.VMEM_SHARED((_NPAD, _DM), jnp.float32),
        pltpu.SemaphoreType.DMA,
        pltpu.SemaphoreType.DMA,
        pltpu.SemaphoreType.DMA,
        pltpu.SemaphoreType.DMA,
    ],
)(_edge_body)


def _head1_body(x_ref, mean_ref, var_ref, w1_ref, b1_ref, a1_ref):
    xn = (x_ref[...] - mean_ref[...]) * jax.lax.rsqrt(var_ref[...] + 1e-8)
    xn = jnp.clip(xn, -5.0, 5.0)
    a1_ref[...] = jnp.tanh(
        jnp.dot(xn, w1_ref[...], preferred_element_type=jnp.float32)
        + b1_ref[...])


def _tc_head1(x, mean2, var2, w1, b12):
    return pl.pallas_call(
        _head1_body,
        grid=(4,),
        in_specs=[
            pl.BlockSpec((_B, _FEAT), lambda j: (0, 0)),
            pl.BlockSpec((1, _FEAT), lambda j: (0, 0)),
            pl.BlockSpec((1, _FEAT), lambda j: (0, 0)),
            pl.BlockSpec((_FEAT, 128), lambda j: (0, j)),
            pl.BlockSpec((1, 128), lambda j: (0, j)),
        ],
        out_specs=pl.BlockSpec((_B, 128), lambda j: (0, j)),
        out_shape=jax.ShapeDtypeStruct((_B, 512), jnp.float32),
    )(x, mean2, var2, w1, b12)


def _head2_body(a1_ref, w2_ref, b2_ref, wv_ref, bv_ref, v_ref):
    a2 = jnp.tanh(
        jnp.dot(a1_ref[...], w2_ref[...], preferred_element_type=jnp.float32)
        + b2_ref[...])
    v_ref[...] = (
        jnp.dot(a2, wv_ref[...], preferred_element_type=jnp.float32)
        + bv_ref[...])


def _tc_head2(a1, w2, b22, wv, bv2):
    return pl.pallas_call(
        _head2_body,
        out_shape=jax.ShapeDtypeStruct((_B, 1), jnp.float32),
    )(a1, w2, b22, wv, bv2)


def kernel(stage_ind, scale_state, sim_obs, edges, W_embed, b_embed, W_msg,
           b_msg, W_upd, b_upd, norm_mean, norm_var, W1, b1, W2, b2, Wv, bv):
    f32 = jnp.float32
    ws = W_msg[:_DH]
    wd = W_msg[_DH:]
    wu1 = W_upd[:_DH]
    wu2 = W_upd[_DH:][jnp.array(_AGG_PERM), :]
    be2 = b_embed.reshape(1, _DH)
    bm2 = b_msg.reshape(1, _DM)
    bu2 = b_upd.reshape(1, _DH)

    x_pad = jnp.pad(sim_obs, ((0, _NPAD - _N), (0, 0)))
    src = jnp.pad(edges[0], (0, _EPAD - _E), constant_values=_N)
    dst = jnp.pad(edges[1], (0, _EPAD - _E), constant_values=_N)
    src = src.reshape(_NTILES, _CHUNKS, _CHUNK)
    dst = dst.reshape(_NTILES, _CHUNKS, _CHUNK)

    h, ms, md = _tc_embed(x_pad, W_embed, be2, ws, wd, bm2)
    for _ in range(_PSTEP):
        agg2 = _edge_kernel(ms, md, src, dst)
        h, ms, md = _tc_update(h, agg2, wu1, wu2, bu2, ws, wd, bm2)

    gnn = h[:_N].reshape(_B, _NPER * _DH)
    x = jnp.concatenate([stage_ind, scale_state, gnn], axis=1)
    a1 = _tc_head1(x, norm_mean.reshape(1, _FEAT), norm_var.reshape(1, _FEAT),
                   W1, b1.reshape(1, 512))
    value = _tc_head2(a1, W2, b2.reshape(1, 256), Wv,
                      bv.reshape(1, 1).astype(f32))
    return value
